# baseline (speedup 1.0000x reference)
import jax
import jax.numpy as jnp
from jax import lax
from jax.experimental import pallas as pl
from jax.experimental.pallas import tpu as pltpu
from jax.experimental.pallas import tpu_sc as plsc

_B = 4096
_S = 200
_D = 32
_N = _B * _S
_NW = 32
_IOCT = _S // 8
_ROWS = 8 * 128


def _xpose_add(G, O, P):
    iota = lax.iota(jnp.int32, 16)

    def m_body(m, _):
        r = m // _D
        d = m - r * _D
        ps = P[r, pl.ds(d * 16, 16)]
        cols = jnp.full((16,), d, jnp.int32)
        rows0 = jnp.full((16,), r * 128, jnp.int32) + iota
        for l0 in range(8):
            v = plsc.load_gather(G, [rows0 + l0 * 16, cols]) + ps
            O[r, d, pl.ds(l0 * 16, 16)] = v
        return ()

    lax.fori_loop(0, 8 * _D, m_body, (), unroll=False)


def _body(x_hbm, tok_hbm, pos_hbm, out_hbm, idx_v, G, O, P, sem_g, sem_o):
    j = lax.axis_index("s") * 2 + lax.axis_index("c")

    def start_gather(i, slot):
        pltpu.sync_copy(
            x_hbm.at[pl.ds((i * _NW + j) * _ROWS, _ROWS)], idx_v[slot])
        return pltpu.async_copy(tok_hbm.at[idx_v[slot]], G[slot], sem_g[slot])

    def start_out(i):
        return [
            pltpu.async_copy(
                O.at[:, pl.ds(8 * ti, 8), :],
                out_hbm.at[pl.ds(8 * i, 8), ti, j],
                sem_o)
            for ti in range(4)
        ]

    gather = [None, None]
    out_cp = None
    gather[0] = start_gather(0, 0)

    for i in range(_IOCT):
        cur = i % 2
        nxt = 1 - cur
        if i + 1 < _IOCT:
            gather[nxt] = start_gather(i + 1, nxt)
        pltpu.sync_copy(pos_hbm.at[pl.ds(8 * i, 8)], P)
        gather[cur].wait()
        if out_cp is not None:
            for cp in out_cp:
                cp.wait()
        _xpose_add(G[cur], O, P)
        out_cp = start_out(i)

    for cp in out_cp:
        cp.wait()


@jax.jit
def _run(x_flat, token_table, posx):
    kcall = pl.kernel(
        _body,
        mesh=plsc.VectorSubcoreMesh(core_axis_name="c", subcore_axis_name="s"),
        out_type=jax.ShapeDtypeStruct((_S, 4, _NW, 8, 128), jnp.float32),
        scratch_types=[
            [pltpu.VMEM((_ROWS,), jnp.int32) for _ in range(2)],
            [pltpu.VMEM((_ROWS, _D), jnp.float32) for _ in range(2)],
            pltpu.VMEM((8, _D, 128), jnp.float32),
            pltpu.VMEM((8, _D * 16), jnp.float32),
            [pltpu.SemaphoreType.DMA for _ in range(2)],
            pltpu.SemaphoreType.DMA,
        ],
        compiler_params=pltpu.CompilerParams(
            use_tc_tiling_on_sc=False, needs_layout_passes=False),
    )
    return kcall(x_flat, token_table, posx)


def kernel(x, token_table, pos_table):
    x_flat = (x.astype(jnp.int32).T
              .reshape(_IOCT, 8, _NW, 128)
              .transpose(0, 2, 1, 3)
              .reshape(_N))
    posx = jnp.broadcast_to(
        pos_table[:_S, :, None], (_S, _D, 16)).reshape(_S, _D * 16)
    out5 = _run(x_flat, token_table, posx)
    return out5.transpose(2, 4, 0, 1, 3).reshape(_B, _S, _D)


# speedup vs baseline: 1.2586x; 1.2586x over previous
import functools

import jax
import jax.numpy as jnp
from jax import lax
from jax.experimental import pallas as pl
from jax.experimental.pallas import tpu as pltpu
from jax.experimental.pallas import tpu_sc as plsc

_B = 4096
_S = 200
_D = 32
_N = _B * _S
_NW = 32
_PER_W = _N // _NW
_CHUNK = 1600
_NCHUNK = _PER_W // _CHUNK
_SEQ_PER_CHUNK = _CHUNK // _S


def _add_pos(rows_v, pos_v):
    def p_body(p, _):
        p0 = pos_v[p, 0:16]
        p1 = pos_v[p, 16:32]
        for k in range(_SEQ_PER_CHUNK):
            r = k * _S + p
            rows_v[r, 0:16] += p0
            rows_v[r, 16:32] += p1
        return ()

    lax.fori_loop(0, _S, p_body, (), unroll=False)


def _body(x_hbm, tok_hbm, pos_hbm, out_hbm, idx_v, rows_v, pos_v,
          sem_i, sem_g, sem_o):
    wid = lax.axis_index("s") * 2 + lax.axis_index("c")
    base = wid * _PER_W

    pltpu.sync_copy(pos_hbm.at[pl.ds(0, _S)], pos_v)

    def off(ci):
        return base + ci * _CHUNK

    pltpu.sync_copy(x_hbm.at[pl.ds(off(0), _CHUNK)], idx_v[0])
    gather = [None, None]
    idx_cp = [None, None]
    out_cp = [None, None]
    gather[0] = pltpu.async_copy(tok_hbm.at[idx_v[0]], rows_v[0], sem_g[0])
    idx_cp[1] = pltpu.async_copy(x_hbm.at[pl.ds(off(1), _CHUNK)], idx_v[1],
                                 sem_i[1])

    for ci in range(_NCHUNK):
        cur = ci % 2
        nxt = 1 - cur
        if ci + 1 < _NCHUNK:
            idx_cp[nxt].wait()
            if out_cp[nxt] is not None:
                out_cp[nxt].wait()
            gather[nxt] = pltpu.async_copy(tok_hbm.at[idx_v[nxt]],
                                           rows_v[nxt], sem_g[nxt])
        gather[cur].wait()
        if ci + 2 < _NCHUNK:
            idx_cp[cur] = pltpu.async_copy(
                x_hbm.at[pl.ds(off(ci + 2), _CHUNK)], idx_v[cur], sem_i[cur])
        _add_pos(rows_v[cur], pos_v)
        out_cp[cur] = pltpu.async_copy(rows_v[cur],
                                       out_hbm.at[pl.ds(off(ci), _CHUNK)],
                                       sem_o[cur])

    out_cp[0].wait()
    out_cp[1].wait()


@jax.jit
def _run(x_flat, token_table, pos_table):
    kcall = pl.kernel(
        _body,
        mesh=plsc.VectorSubcoreMesh(core_axis_name="c", subcore_axis_name="s"),
        out_type=jax.ShapeDtypeStruct((_N, _D), jnp.float32),
        scratch_types=[
            [pltpu.VMEM((_CHUNK,), jnp.int32) for _ in range(2)],
            [pltpu.VMEM((_CHUNK, _D), jnp.float32) for _ in range(2)],
            pltpu.VMEM((_S, _D), jnp.float32),
            [pltpu.SemaphoreType.DMA for _ in range(2)],
            [pltpu.SemaphoreType.DMA for _ in range(2)],
            [pltpu.SemaphoreType.DMA for _ in range(2)],
        ],
        compiler_params=pltpu.CompilerParams(use_tc_tiling_on_sc=False),
    )
    return kcall(x_flat, token_table, pos_table)


def kernel(x, token_table, pos_table):
    x_flat = x.reshape(_N).astype(jnp.int32)
    out = _run(x_flat, token_table, pos_table)
    return out.reshape(_B, _S, _D)
